# baseline (speedup 1.0000x reference)
import functools

import jax
import jax.numpy as jnp
from jax import lax
from jax.experimental import pallas as pl
from jax.experimental.pallas import tpu as pltpu
from jax.experimental.pallas import tpu_sc as plsc

V, D, B = 1000000, 64, 16384
NC, NS = 2, 16
NW = NC * NS
B_PER_W = B // NW
CH = 128
NCH = B_PER_W // CH

_mesh = plsc.VectorSubcoreMesh(core_axis_name="c", subcore_axis_name="s")


@functools.partial(
    pl.kernel,
    mesh=_mesh,
    out_type=jax.ShapeDtypeStruct((B, D), jnp.float32),
    scratch_types=[
        pltpu.VMEM((NCH, CH), jnp.int32),
        pltpu.VMEM((CH, 2 * D), jnp.float32),
        pltpu.VMEM((CH, D), jnp.float32),
        pltpu.SemaphoreType.DMA,
    ],
    compiler_params=pltpu.CompilerParams(use_tc_tiling_on_sc=False),
)
def _gather_sc(tablep, idx_hbm, out_hbm, idx_v, rows_v, out_v, sem):
    wid = lax.axis_index("s") * NC + lax.axis_index("c")
    base = wid * B_PER_W
    pltpu.sync_copy(idx_hbm.at[wid], idx_v)
    L = 16
    for j in range(NCH):
        pltpu.async_copy(tablep.at[idx_v.at[j]], rows_v, sem).wait()

        @pl.loop(0, CH)
        def _(i):
            for c in range(D // L):
                out_v[i, pl.ds(c * L, L)] = rows_v[i, pl.ds(c * L, L)]

        pltpu.sync_copy(out_v, out_hbm.at[pl.ds(base + j * CH, CH)])


def kernel(input, indices):
    idx = indices.astype(jnp.int32).reshape(NW, NCH, CH)
    tablep = jnp.pad(input, ((0, 0), (0, D)))
    return _gather_sc(tablep, idx)


# speedup vs baseline: 1.2056x; 1.2056x over previous
import functools

import jax
import jax.numpy as jnp
from jax import lax
from jax.experimental import pallas as pl
from jax.experimental.pallas import tpu as pltpu
from jax.experimental.pallas import tpu_sc as plsc

V, D, B = 1000000, 64, 16384
NC, NS = 2, 16
NW = NC * NS
B_PER_W = B // NW
HALF = B_PER_W // 2
NSEM = 4

_mesh = plsc.VectorSubcoreMesh(core_axis_name="c", subcore_axis_name="s")


@functools.partial(
    pl.kernel,
    mesh=_mesh,
    out_type=jax.ShapeDtypeStruct((B, D), jnp.float32),
    scratch_types=[
        pltpu.VMEM((B_PER_W,), jnp.int32),
        pltpu.VMEM((B_PER_W,), jnp.int32),
        pltpu.VMEM((HALF, D), jnp.float32),
    ] + [pltpu.SemaphoreType.DMA] * NSEM,
)
def _gather_sc(table_hbm, bidx_hbm, sidx_hbm, out_hbm,
               bidx_v, sidx_v, rows_v, *sems):
    wid = lax.axis_index("s") * NC + lax.axis_index("c")
    base = wid * B_PER_W
    pltpu.sync_copy(bidx_hbm.at[wid], bidx_v)
    pltpu.sync_copy(sidx_hbm.at[wid], sidx_v)

    @pl.loop(0, HALF // 16)
    def _(g):
        b_vec = bidx_v[pl.ds(g * 16, 16)]
        s_vec = sidx_v[pl.ds(g * 16, 16)]
        for j in range(16):
            pltpu.async_copy(
                table_hbm.at[b_vec[j], s_vec[j]],
                out_hbm.at[base + g * 16 + j],
                sems[j % NSEM],
            )

    @pl.loop(0, HALF // 16)
    def _(g):
        b_vec = bidx_v[pl.ds(HALF + g * 16, 16)]
        s_vec = sidx_v[pl.ds(HALF + g * 16, 16)]
        for j in range(16):
            pltpu.sync_copy(
                table_hbm.at[b_vec[j], s_vec[j]],
                rows_v.at[g * 16 + j],
            )

    pltpu.sync_copy(rows_v, out_hbm.at[pl.ds(base + HALF, HALF)])

    for k in range(NSEM):
        pltpu.make_async_copy(
            out_hbm.at[pl.ds(base, HALF // NSEM)],
            out_hbm.at[pl.ds(base, HALF // NSEM)],
            sems[k],
        ).wait()


def kernel(input, indices):
    idx = indices.astype(jnp.int32)
    table3 = input.reshape(V // 8, 8, D)
    bidx = (idx >> 3).reshape(NW, B_PER_W)
    sidx = (idx & 7).reshape(NW, B_PER_W)
    return _gather_sc(table3, bidx, sidx)
